# baseline (speedup 1.0000x reference)
import functools

import jax
import jax.numpy as jnp
from jax import lax
from jax.experimental import pallas as pl
from jax.experimental.pallas import tpu as pltpu
from jax.experimental.pallas import tpu_sc as plsc

NC = 2
NS = 16
NW = NC * NS


def _deg_body(n, ch, k, dst_hbm, z_hbm, deg_hbm, idx_v, hist_v, iota_v,
              hist_sh):
    nr = n // 128
    c = lax.axis_index("c")
    s = lax.axis_index("s")
    wid = c * NS + s
    pltpu.sync_copy(dst_hbm.at[wid], idx_v)

    def zb(j, carry):
        for i in range(8):
            hist_v[j, pl.ds(16 * i, 16)] = jnp.zeros((16,), jnp.float32)
        return carry

    lax.fori_loop(0, nr, zb, 0)
    for i in range(nr // 16):
        iota_v[0, pl.ds(16 * i, 16)] = lax.iota(jnp.int32, 16) + 16 * i

    @pl.when(s == 0)
    def _():
        pltpu.sync_copy(z_hbm, hist_sh)

    ones = jnp.ones((16,), jnp.float32)

    def body(j, carry):
        for i in range(k // 16):
            idx = idx_v[j, pl.ds(16 * i, 16)]
            row = lax.shift_right_logical(idx, 7)
            col = lax.bitwise_and(idx, 127)
            plsc.addupdate_scatter(hist_v, [row, col], ones)
        return carry

    lax.fori_loop(0, ch, body, 0)
    plsc.subcore_barrier()
    pltpu.sync_copy(hist_v, hist_sh.at[iota_v.at[0]], add=True)
    plsc.subcore_barrier()
    nd = n // 1024

    @pl.when(s < nd)
    def _():
        pltpu.sync_copy(hist_sh.at[pl.ds(s * 8, 8)],
                        deg_hbm.at[c, pl.ds(s * 8, 8)])


def _make_deg(n, ch, k):
    mesh = plsc.VectorSubcoreMesh(core_axis_name="c", subcore_axis_name="s")
    nr = n // 128
    return pl.kernel(
        functools.partial(_deg_body, n, ch, k),
        out_type=jax.ShapeDtypeStruct((NC, nr, 128), jnp.float32),
        mesh=mesh,
        scratch_types=[
            pltpu.VMEM((ch, k), jnp.int32),
            pltpu.VMEM((nr, 128), jnp.float32),
            pltpu.VMEM((1, nr), jnp.int32),
            pltpu.VMEM_SHARED((nr, 128), jnp.float32),
        ],
        compiler_params=pltpu.CompilerParams(needs_layout_passes=False),
    )


def _scat_body(n, h, nb, cb, k, src_hbm, dst_hbm, hp_hbm, z_hbm, acc_hbm,
               isrc_v, idst_v, rows0_v, rows1_v, acc_sh, sem0, sem1):
    rp = n // NS
    c = lax.axis_index("c")
    s = lax.axis_index("s")
    wid = c * NS + s
    pltpu.sync_copy(z_hbm, acc_sh.at[pl.ds(s * rp, rp)])
    plsc.subcore_barrier()

    for blk in range(nb):
        pltpu.sync_copy(src_hbm.at[wid, blk], isrc_v)
        pltpu.sync_copy(dst_hbm.at[wid, blk], idst_v)
        pltpu.async_copy(hp_hbm.at[isrc_v.at[0]], rows0_v, sem0)

        def body(jj, carry):
            j0 = 2 * jj
            j1 = j0 + 1
            j2 = j0 + 2
            pltpu.async_copy(hp_hbm.at[isrc_v.at[j1]], rows1_v, sem1)
            pltpu.make_async_copy(hp_hbm.at[isrc_v.at[j0]], rows0_v,
                                  sem0).wait()
            pltpu.sync_copy(rows0_v, acc_sh.at[idst_v.at[j0]], add=True)

            @pl.when(j2 < cb)
            def _():
                pltpu.async_copy(hp_hbm.at[isrc_v.at[j2]], rows0_v, sem0)

            pltpu.make_async_copy(hp_hbm.at[isrc_v.at[j1]], rows1_v,
                                  sem1).wait()
            pltpu.sync_copy(rows1_v, acc_sh.at[idst_v.at[j1]], add=True)
            return carry

        lax.fori_loop(0, cb // 2, body, 0)

    plsc.subcore_barrier()
    pltpu.sync_copy(acc_sh.at[pl.ds(s * rp, rp)],
                    acc_hbm.at[c, pl.ds(s * rp, rp)])


def _make_scat(n, h, nb, cb, k):
    mesh = plsc.VectorSubcoreMesh(core_axis_name="c", subcore_axis_name="s")
    return pl.kernel(
        functools.partial(_scat_body, n, h, nb, cb, k),
        out_type=jax.ShapeDtypeStruct((NC, n, h), jnp.float32),
        mesh=mesh,
        scratch_types=[
            pltpu.VMEM((cb, k), jnp.int32),
            pltpu.VMEM((cb, k), jnp.int32),
            pltpu.VMEM((k, h), jnp.float32),
            pltpu.VMEM((k, h), jnp.float32),
            pltpu.VMEM_SHARED((n, h), jnp.float32),
            pltpu.SemaphoreType.DMA,
            pltpu.SemaphoreType.DMA,
        ],
    )


def _dinv3(deg_ref):
    deg = deg_ref[0] + deg_ref[1] + 1.0
    return lax.rsqrt(deg)[:, :, None]


def _hp_body(deg_ref, x_ref, w_ref, hp_ref):
    br = x_ref.shape[0]
    xw = jnp.dot(x_ref[...], w_ref[...], preferred_element_type=jnp.float32,
                 precision=lax.Precision.HIGHEST)
    xw3 = xw.reshape(br // 128, 128, 128)
    hp_ref[...] = (_dinv3(deg_ref) * xw3).reshape(br, 128)


def _fin_body(nblk, deg_ref, acc_ref, hp_ref, wp_ref, cond_ref, wgam_ref,
              bgam_ref, wbet_ref, bbet_ref, bgcn_ref, bproj_ref, wout_ref,
              bout_ref, out_ref, m_acc):
    i = pl.program_id(0)

    @pl.when(i == 0)
    def _():
        m_acc[...] = jnp.zeros_like(m_acc)

    br = hp_ref.shape[0]
    t = acc_ref[0] + acc_ref[1] + hp_ref[...]
    g3 = _dinv3(deg_ref) * t.reshape(br // 128, 128, 128)
    g = g3.reshape(br, 128) + bgcn_ref[...]
    r = jnp.maximum(g, 0.0)
    m_acc[...] += lax.dot_general(
        r, wp_ref[...], (((0,), (0,)), ((), ())),
        preferred_element_type=jnp.float32, precision=lax.Precision.HIGHEST)

    @pl.when(i == nblk - 1)
    def _():
        hi = lax.Precision.HIGHEST
        cond = cond_ref[...]
        gamma = jnp.dot(cond, wgam_ref[...], preferred_element_type=jnp.float32,
                        precision=hi) + bgam_ref[...]
        beta = jnp.dot(cond, wbet_ref[...], preferred_element_type=jnp.float32,
                       precision=hi) + bbet_ref[...]
        z = jnp.dot(gamma, m_acc[...], preferred_element_type=jnp.float32,
                    precision=hi) + bproj_ref[...] + beta
        z = jnp.maximum(z, 0.0)
        o = jnp.dot(z, wout_ref[...], preferred_element_type=jnp.float32,
                    precision=hi) + bout_ref[...]
        mx = jnp.max(o, axis=1, keepdims=True)
        lse = mx + jnp.log(jnp.sum(jnp.exp(o - mx), axis=1, keepdims=True))
        out_ref[...] = o - lse


def kernel(x, edge_index, condition, W_gcn, b_gcn, W_gamma, b_gamma, W_beta,
           b_beta, W_proj, b_proj, W_out, b_out):
    n, d = x.shape
    h = W_gcn.shape[1]
    e = edge_index.shape[1]
    b, c = condition.shape

    nblk = 10
    np_ = 10240
    assert n <= np_ and np_ % (NS * 8) == 0 and np_ % nblk == 0
    pad = np_ - n
    xp = jnp.pad(x, ((0, pad), (0, 0)))
    wpp = jnp.pad(W_proj, ((0, pad), (0, 0)))

    k = 128
    nb, cb = 5, 16
    ep = NW * nb * cb * k
    epad = ep - e
    assert epad >= 0 and pad > 0
    fill_src = n + (jnp.arange(epad, dtype=jnp.int32) % pad)
    fill_dst = n + ((jnp.arange(epad, dtype=jnp.int32) + 7) % pad)
    srcr = jnp.concatenate([edge_index[0], fill_src]).reshape(NW, nb, cb, k)
    dstr = jnp.concatenate([edge_index[1], fill_dst]).reshape(NW, nb, cb, k)
    rp = np_ // NS
    nr = np_ // 128
    z_deg = jnp.zeros((nr, 128), jnp.float32)
    z_acc = jnp.zeros((rp, h), jnp.float32)

    dstr_deg = dstr.reshape(NW, nb * cb, k)
    deg2d = _make_deg(np_, nb * cb, k)(dstr_deg, z_deg)

    br = np_ // nblk
    dr = br // 128
    hp = pl.pallas_call(
        _hp_body,
        grid=(nblk,),
        in_specs=[
            pl.BlockSpec((NC, dr, 128), lambda i: (0, i, 0)),
            pl.BlockSpec((br, d), lambda i: (i, 0)),
            pl.BlockSpec((d, h), lambda i: (0, 0)),
        ],
        out_specs=pl.BlockSpec((br, h), lambda i: (i, 0)),
        out_shape=jax.ShapeDtypeStruct((np_, h), jnp.float32),
    )(deg2d, xp, W_gcn)

    accs = _make_scat(np_, h, nb, cb, k)(srcr, dstr, hp, z_acc)

    cst = lambda shape: pl.BlockSpec(shape, lambda i: tuple(0 for _ in shape))
    out = pl.pallas_call(
        functools.partial(_fin_body, nblk),
        grid=(nblk,),
        in_specs=[
            pl.BlockSpec((NC, dr, 128), lambda i: (0, i, 0)),
            pl.BlockSpec((NC, br, h), lambda i: (0, i, 0)),
            pl.BlockSpec((br, h), lambda i: (i, 0)),
            pl.BlockSpec((br, h), lambda i: (i, 0)),
            cst((b, c)),
            cst((c, h)),
            cst((1, h)),
            cst((c, h)),
            cst((1, h)),
            cst((1, h)),
            cst((1, h)),
            cst((h, 2)),
            cst((1, 2)),
        ],
        out_specs=pl.BlockSpec((b, 2), lambda i: (0, 0)),
        out_shape=jax.ShapeDtypeStruct((b, 2), jnp.float32),
        scratch_shapes=[pltpu.VMEM((h, h), jnp.float32)],
        compiler_params=pltpu.CompilerParams(
            dimension_semantics=("arbitrary",)),
    )(deg2d, accs, hp, wpp, condition, W_gamma, b_gamma.reshape(1, h),
      W_beta, b_beta.reshape(1, h), b_gcn.reshape(1, h), b_proj.reshape(1, h),
      W_out, b_out.reshape(1, 2))
    return out


# speedup vs baseline: 1.0495x; 1.0495x over previous
import functools

import jax
import jax.numpy as jnp
from jax import lax
from jax.experimental import pallas as pl
from jax.experimental.pallas import tpu as pltpu
from jax.experimental.pallas import tpu_sc as plsc

NC = 2
NS = 16
NW = NC * NS


def _deg_body(n, ch, k, dst_hbm, z_hbm, deg_hbm, idx_v, hist_v, iota_v,
              hist_sh):
    nr = n // 128
    c = lax.axis_index("c")
    s = lax.axis_index("s")
    wid = c * NS + s
    pltpu.sync_copy(dst_hbm.at[wid], idx_v)

    def zb(j, carry):
        for i in range(8):
            hist_v[j, pl.ds(16 * i, 16)] = jnp.zeros((16,), jnp.float32)
        return carry

    lax.fori_loop(0, nr, zb, 0)
    for i in range(nr // 16):
        iota_v[0, pl.ds(16 * i, 16)] = lax.iota(jnp.int32, 16) + 16 * i

    @pl.when(s == 0)
    def _():
        pltpu.sync_copy(z_hbm, hist_sh)

    ones = jnp.ones((16,), jnp.float32)

    def body(j, carry):
        for i in range(k // 16):
            idx = idx_v[j, pl.ds(16 * i, 16)]
            row = lax.shift_right_logical(idx, 7)
            col = lax.bitwise_and(idx, 127)
            plsc.addupdate_scatter(hist_v, [row, col], ones)
        return carry

    lax.fori_loop(0, ch, body, 0)
    plsc.subcore_barrier()
    pltpu.sync_copy(hist_v, hist_sh.at[iota_v.at[0]], add=True)
    plsc.subcore_barrier()
    nd = n // 1024

    @pl.when(s < nd)
    def _():
        pltpu.sync_copy(hist_sh.at[pl.ds(s * 8, 8)],
                        deg_hbm.at[c, pl.ds(s * 8, 8)])


def _make_deg(n, ch, k):
    mesh = plsc.VectorSubcoreMesh(core_axis_name="c", subcore_axis_name="s")
    nr = n // 128
    return pl.kernel(
        functools.partial(_deg_body, n, ch, k),
        out_type=jax.ShapeDtypeStruct((NC, nr, 128), jnp.float32),
        mesh=mesh,
        scratch_types=[
            pltpu.VMEM((ch, k), jnp.int32),
            pltpu.VMEM((nr, 128), jnp.float32),
            pltpu.VMEM((1, nr), jnp.int32),
            pltpu.VMEM_SHARED((nr, 128), jnp.float32),
        ],
        compiler_params=pltpu.CompilerParams(needs_layout_passes=False),
    )


def _scat_body(n, h, nb, cb, k, src_hbm, dst_hbm, hp_hbm, acc_hbm,
               isrc_v, idst_v, rows0_v, rows1_v, acc_sh, sem0, sem1):
    rp = n // NS
    c = lax.axis_index("c")
    s = lax.axis_index("s")
    wid = c * NS + s

    def zb(j, carry):
        for i in range(h // 16):
            rows0_v[j, pl.ds(16 * i, 16)] = jnp.zeros((16,), jnp.float32)
        return carry

    lax.fori_loop(0, k, zb, 0)
    for j in range(rp // k):
        pltpu.sync_copy(rows0_v, acc_sh.at[pl.ds(s * rp + j * k, k)])
    plsc.subcore_barrier()

    for blk in range(nb):
        pltpu.sync_copy(src_hbm.at[wid, blk], isrc_v)
        pltpu.sync_copy(dst_hbm.at[wid, blk], idst_v)
        pltpu.async_copy(hp_hbm.at[isrc_v.at[0]], rows0_v, sem0)

        def body(jj, carry):
            j0 = 2 * jj
            j1 = j0 + 1
            j2 = j0 + 2
            pltpu.async_copy(hp_hbm.at[isrc_v.at[j1]], rows1_v, sem1)
            pltpu.make_async_copy(hp_hbm.at[isrc_v.at[j0]], rows0_v,
                                  sem0).wait()
            pltpu.sync_copy(rows0_v, acc_sh.at[idst_v.at[j0]], add=True)

            @pl.when(j2 < cb)
            def _():
                pltpu.async_copy(hp_hbm.at[isrc_v.at[j2]], rows0_v, sem0)

            pltpu.make_async_copy(hp_hbm.at[isrc_v.at[j1]], rows1_v,
                                  sem1).wait()
            pltpu.sync_copy(rows1_v, acc_sh.at[idst_v.at[j1]], add=True)
            return carry

        lax.fori_loop(0, cb // 2, body, 0)

    plsc.subcore_barrier()
    pltpu.sync_copy(acc_sh.at[pl.ds(s * rp, rp)],
                    acc_hbm.at[c, pl.ds(s * rp, rp)])


def _make_scat(n, h, nb, cb, k):
    mesh = plsc.VectorSubcoreMesh(core_axis_name="c", subcore_axis_name="s")
    return pl.kernel(
        functools.partial(_scat_body, n, h, nb, cb, k),
        out_type=jax.ShapeDtypeStruct((NC, n, h), jnp.float32),
        mesh=mesh,
        scratch_types=[
            pltpu.VMEM((cb, k), jnp.int32),
            pltpu.VMEM((cb, k), jnp.int32),
            pltpu.VMEM((k, h), jnp.float32),
            pltpu.VMEM((k, h), jnp.float32),
            pltpu.VMEM_SHARED((n, h), jnp.float32),
            pltpu.SemaphoreType.DMA,
            pltpu.SemaphoreType.DMA,
        ],
    )


def _dinv3(deg_ref):
    deg = deg_ref[0] + deg_ref[1] + 1.0
    return lax.rsqrt(deg)[:, :, None]


def _hp_body(deg_ref, x_ref, w_ref, hp_ref):
    br = x_ref.shape[0]
    xw = jnp.dot(x_ref[...], w_ref[...], preferred_element_type=jnp.float32,
                 precision=lax.Precision.HIGHEST)
    xw3 = xw.reshape(br // 128, 128, 128)
    hp_ref[...] = (_dinv3(deg_ref) * xw3).reshape(br, 128)


def _fin_body(nblk, nvalid, deg_ref, acc_ref, hp_ref, wp_ref, cond_ref, wgam_ref,
              bgam_ref, wbet_ref, bbet_ref, bgcn_ref, bproj_ref, wout_ref,
              bout_ref, out_ref, m_acc):
    i = pl.program_id(0)

    @pl.when(i == 0)
    def _():
        m_acc[...] = jnp.zeros_like(m_acc)

    br = hp_ref.shape[0]
    t = acc_ref[0] + acc_ref[1] + hp_ref[...]
    g3 = _dinv3(deg_ref) * t.reshape(br // 128, 128, 128)
    g = g3.reshape(br, 128) + bgcn_ref[...]
    r = jnp.maximum(g, 0.0)
    rows = lax.broadcasted_iota(jnp.int32, (br, 1), 0) + i * br
    valid = rows < nvalid
    r = jnp.where(valid, r, 0.0)
    wp = jnp.where(valid, wp_ref[...], 0.0)
    m_acc[...] += lax.dot_general(
        r, wp, (((0,), (0,)), ((), ())),
        preferred_element_type=jnp.float32, precision=lax.Precision.HIGHEST)

    @pl.when(i == nblk - 1)
    def _():
        hi = lax.Precision.HIGHEST
        cond = cond_ref[...]
        gamma = jnp.dot(cond, wgam_ref[...], preferred_element_type=jnp.float32,
                        precision=hi) + bgam_ref[...]
        beta = jnp.dot(cond, wbet_ref[...], preferred_element_type=jnp.float32,
                       precision=hi) + bbet_ref[...]
        z = jnp.dot(gamma, m_acc[...], preferred_element_type=jnp.float32,
                    precision=hi) + bproj_ref[...] + beta
        z = jnp.maximum(z, 0.0)
        o = jnp.dot(z, wout_ref[...], preferred_element_type=jnp.float32,
                    precision=hi) + bout_ref[...]
        mx = jnp.max(o, axis=1, keepdims=True)
        lse = mx + jnp.log(jnp.sum(jnp.exp(o - mx), axis=1, keepdims=True))
        out_ref[...] = o - lse


def kernel(x, edge_index, condition, W_gcn, b_gcn, W_gamma, b_gamma, W_beta,
           b_beta, W_proj, b_proj, W_out, b_out):
    n, d = x.shape
    h = W_gcn.shape[1]
    e = edge_index.shape[1]
    b, c = condition.shape

    nblk = 10
    np_ = 10240
    assert n <= np_ and np_ % (NS * 8) == 0 and np_ % nblk == 0
    pad = np_ - n

    k = 128
    nb, cb = 5, 16
    ep = NW * nb * cb * k
    epad = ep - e
    assert epad >= 0 and pad > 0
    fill_src = n + (jnp.arange(epad, dtype=jnp.int32) % pad)
    fill_dst = n + ((jnp.arange(epad, dtype=jnp.int32) + 7) % pad)
    srcr = jnp.concatenate([edge_index[0], fill_src]).reshape(NW, nb, cb, k)
    dstr = jnp.concatenate([edge_index[1], fill_dst]).reshape(NW, nb, cb, k)
    rp = np_ // NS
    nr = np_ // 128
    z_deg = jnp.zeros((nr, 128), jnp.float32)

    dstr_deg = dstr.reshape(NW, nb * cb, k)
    deg2d = _make_deg(np_, nb * cb, k)(dstr_deg, z_deg)

    br = np_ // nblk
    dr = br // 128
    hp = pl.pallas_call(
        _hp_body,
        grid=(nblk,),
        in_specs=[
            pl.BlockSpec((NC, dr, 128), lambda i: (0, i, 0)),
            pl.BlockSpec((br, d), lambda i: (i, 0)),
            pl.BlockSpec((d, h), lambda i: (0, 0)),
        ],
        out_specs=pl.BlockSpec((br, h), lambda i: (i, 0)),
        out_shape=jax.ShapeDtypeStruct((np_, h), jnp.float32),
    )(deg2d, x, W_gcn)

    accs = _make_scat(np_, h, nb, cb, k)(srcr, dstr, hp)

    cst = lambda shape: pl.BlockSpec(shape, lambda i: tuple(0 for _ in shape))
    out = pl.pallas_call(
        functools.partial(_fin_body, nblk, n),
        grid=(nblk,),
        in_specs=[
            pl.BlockSpec((NC, dr, 128), lambda i: (0, i, 0)),
            pl.BlockSpec((NC, br, h), lambda i: (0, i, 0)),
            pl.BlockSpec((br, h), lambda i: (i, 0)),
            pl.BlockSpec((br, h), lambda i: (i, 0)),
            cst((b, c)),
            cst((c, h)),
            cst((1, h)),
            cst((c, h)),
            cst((1, h)),
            cst((1, h)),
            cst((1, h)),
            cst((h, 2)),
            cst((1, 2)),
        ],
        out_specs=pl.BlockSpec((b, 2), lambda i: (0, 0)),
        out_shape=jax.ShapeDtypeStruct((b, 2), jnp.float32),
        scratch_shapes=[pltpu.VMEM((h, h), jnp.float32)],
        compiler_params=pltpu.CompilerParams(
            dimension_semantics=("arbitrary",)),
    )(deg2d, accs, hp, W_proj, condition, W_gamma, b_gamma.reshape(1, h),
      W_beta, b_beta.reshape(1, h), b_gcn.reshape(1, h), b_proj.reshape(1, h),
      W_out, b_out.reshape(1, 2))
    return out


# speedup vs baseline: 1.1027x; 1.0508x over previous
import functools

import jax
import jax.numpy as jnp
from jax import lax
from jax.experimental import pallas as pl
from jax.experimental.pallas import tpu as pltpu
from jax.experimental.pallas import tpu_sc as plsc

NC = 2
NS = 16
NW = NC * NS


def _deg_body(n, ch, k, dst_hbm, z_hbm, deg_hbm, idx_v, hist_v, iota_v,
              hist_sh):
    nr = n // 128
    c = lax.axis_index("c")
    s = lax.axis_index("s")
    wid = c * NS + s
    pltpu.sync_copy(dst_hbm.at[wid], idx_v)

    def zb(j, carry):
        for i in range(8):
            hist_v[j, pl.ds(16 * i, 16)] = jnp.zeros((16,), jnp.float32)
        return carry

    lax.fori_loop(0, nr, zb, 0)
    for i in range(nr // 16):
        iota_v[0, pl.ds(16 * i, 16)] = lax.iota(jnp.int32, 16) + 16 * i

    @pl.when(s == 0)
    def _():
        pltpu.sync_copy(z_hbm, hist_sh)

    ones = jnp.ones((16,), jnp.float32)

    def body(j, carry):
        for i in range(k // 16):
            idx = idx_v[j, pl.ds(16 * i, 16)]
            row = lax.shift_right_logical(idx, 7)
            col = lax.bitwise_and(idx, 127)
            plsc.addupdate_scatter(hist_v, [row, col], ones)
        return carry

    lax.fori_loop(0, ch, body, 0)
    plsc.subcore_barrier()
    pltpu.sync_copy(hist_v, hist_sh.at[iota_v.at[0]], add=True)
    plsc.subcore_barrier()
    nd = n // 1024

    @pl.when(s < nd)
    def _():
        pltpu.sync_copy(hist_sh.at[pl.ds(s * 8, 8)],
                        deg_hbm.at[c, pl.ds(s * 8, 8)])


def _make_deg(n, ch, k):
    mesh = plsc.VectorSubcoreMesh(core_axis_name="c", subcore_axis_name="s")
    nr = n // 128
    return pl.kernel(
        functools.partial(_deg_body, n, ch, k),
        out_type=jax.ShapeDtypeStruct((NC, nr, 128), jnp.float32),
        mesh=mesh,
        scratch_types=[
            pltpu.VMEM((ch, k), jnp.int32),
            pltpu.VMEM((nr, 128), jnp.float32),
            pltpu.VMEM((1, nr), jnp.int32),
            pltpu.VMEM_SHARED((nr, 128), jnp.float32),
        ],
        compiler_params=pltpu.CompilerParams(needs_layout_passes=False),
    )


def _scat_body(n, h, nb, cb, k, src_hbm, dst_hbm, hp_hbm, acc_hbm,
               isrc_v, idst_v, rows0_v, rows1_v, acc_sh, sem0, sem1):
    rp = n // NS
    c = lax.axis_index("c")
    s = lax.axis_index("s")
    wid = c * NS + s

    def zb(j, carry):
        for i in range(h // 16):
            rows0_v[j, pl.ds(16 * i, 16)] = jnp.zeros((16,), jnp.float32)
        return carry

    lax.fori_loop(0, k, zb, 0)
    for j in range(rp // k):
        pltpu.sync_copy(rows0_v, acc_sh.at[pl.ds(s * rp + j * k, k)])
    plsc.subcore_barrier()

    for blk in range(nb):
        pltpu.sync_copy(src_hbm.at[wid, blk], isrc_v)
        pltpu.sync_copy(dst_hbm.at[wid, blk], idst_v)
        pltpu.async_copy(hp_hbm.at[isrc_v.at[0]], rows0_v, sem0)

        def body(jj, carry):
            j0 = 2 * jj
            j1 = j0 + 1
            j2 = j0 + 2
            pltpu.async_copy(hp_hbm.at[isrc_v.at[j1]], rows1_v, sem1)
            pltpu.make_async_copy(hp_hbm.at[isrc_v.at[j0]], rows0_v,
                                  sem0).wait()
            pltpu.sync_copy(rows0_v, acc_sh.at[idst_v.at[j0]], add=True)

            @pl.when(j2 < cb)
            def _():
                pltpu.async_copy(hp_hbm.at[isrc_v.at[j2]], rows0_v, sem0)

            pltpu.make_async_copy(hp_hbm.at[isrc_v.at[j1]], rows1_v,
                                  sem1).wait()
            pltpu.sync_copy(rows1_v, acc_sh.at[idst_v.at[j1]], add=True)
            return carry

        lax.fori_loop(0, cb // 2, body, 0)

    plsc.subcore_barrier()
    pltpu.sync_copy(acc_sh.at[pl.ds(s * rp, rp)],
                    acc_hbm.at[c, pl.ds(s * rp, rp)])


def _make_scat(n, h, nb, cb, k):
    mesh = plsc.VectorSubcoreMesh(core_axis_name="c", subcore_axis_name="s")
    return pl.kernel(
        functools.partial(_scat_body, n, h, nb, cb, k),
        out_type=jax.ShapeDtypeStruct((NC, n, h), jnp.float32),
        mesh=mesh,
        scratch_types=[
            pltpu.VMEM((cb, k), jnp.int32),
            pltpu.VMEM((cb, k), jnp.int32),
            pltpu.VMEM((k, h), jnp.float32),
            pltpu.VMEM((k, h), jnp.float32),
            pltpu.VMEM_SHARED((n, h), jnp.float32),
            pltpu.SemaphoreType.DMA,
            pltpu.SemaphoreType.DMA,
        ],
    )


def _dinv3(deg_ref):
    deg = deg_ref[0] + deg_ref[1] + 1.0
    return lax.rsqrt(deg)[:, :, None]


def _hp_body(deg_ref, x_ref, w_ref, hp_ref):
    br = x_ref.shape[0]
    xw = jnp.dot(x_ref[...], w_ref[...], preferred_element_type=jnp.float32,
                 precision=lax.Precision.HIGHEST)
    xw3 = xw.reshape(br // 128, 128, 128)
    hp_ref[...] = (_dinv3(deg_ref) * xw3).reshape(br, 128)


def _fin_body(nblk, nvalid, deg_ref, acc_ref, hp_ref, wp_ref, cond_ref, wgam_ref,
              bgam_ref, wbet_ref, bbet_ref, bgcn_ref, bproj_ref, wout_ref,
              bout_ref, out_ref, m_acc):
    i = pl.program_id(0)

    @pl.when(i == 0)
    def _():
        m_acc[...] = jnp.zeros_like(m_acc)

    br = hp_ref.shape[0]
    t = acc_ref[0] + acc_ref[1] + hp_ref[...]
    g3 = _dinv3(deg_ref) * t.reshape(br // 128, 128, 128)
    g = g3.reshape(br, 128) + bgcn_ref[...]
    r = jnp.maximum(g, 0.0)
    rows = lax.broadcasted_iota(jnp.int32, (br, 1), 0) + i * br
    valid = rows < nvalid
    r = jnp.where(valid, r, 0.0)
    wp = jnp.where(valid, wp_ref[...], 0.0)
    m_acc[...] += lax.dot_general(
        r, wp, (((0,), (0,)), ((), ())),
        preferred_element_type=jnp.float32, precision=lax.Precision.HIGHEST)

    @pl.when(i == nblk - 1)
    def _():
        hi = lax.Precision.HIGHEST
        cond = cond_ref[...]
        gamma = jnp.dot(cond, wgam_ref[...], preferred_element_type=jnp.float32,
                        precision=hi) + bgam_ref[...]
        beta = jnp.dot(cond, wbet_ref[...], preferred_element_type=jnp.float32,
                       precision=hi) + bbet_ref[...]
        z = jnp.dot(gamma, m_acc[...], preferred_element_type=jnp.float32,
                    precision=hi) + bproj_ref[...] + beta
        z = jnp.maximum(z, 0.0)
        o = jnp.dot(z, wout_ref[...], preferred_element_type=jnp.float32,
                    precision=hi) + bout_ref[...]
        mx = jnp.max(o, axis=1, keepdims=True)
        lse = mx + jnp.log(jnp.sum(jnp.exp(o - mx), axis=1, keepdims=True))
        out_ref[...] = o - lse


def kernel(x, edge_index, condition, W_gcn, b_gcn, W_gamma, b_gamma, W_beta,
           b_beta, W_proj, b_proj, W_out, b_out):
    n, d = x.shape
    h = W_gcn.shape[1]
    e = edge_index.shape[1]
    b, c = condition.shape

    nblk = 10
    np_ = 10240
    assert n <= np_ and np_ % (NS * 8) == 0 and np_ % nblk == 0
    pad = np_ - n

    k = 128
    nb, cb = 2, 40
    ep = NW * nb * cb * k
    epad = ep - e
    assert epad >= 0 and pad > 0
    fill_src = n + (jnp.arange(epad, dtype=jnp.int32) % pad)
    fill_dst = n + ((jnp.arange(epad, dtype=jnp.int32) + 7) % pad)
    srcr = jnp.concatenate([edge_index[0], fill_src]).reshape(NW, nb, cb, k)
    dstr = jnp.concatenate([edge_index[1], fill_dst]).reshape(NW, nb, cb, k)
    rp = np_ // NS
    nr = np_ // 128
    z_deg = jnp.zeros((nr, 128), jnp.float32)

    dstr_deg = dstr.reshape(NW, nb * cb, k)
    deg2d = _make_deg(np_, nb * cb, k)(dstr_deg, z_deg)

    br = np_ // nblk
    dr = br // 128
    hp = pl.pallas_call(
        _hp_body,
        grid=(nblk,),
        in_specs=[
            pl.BlockSpec((NC, dr, 128), lambda i: (0, i, 0)),
            pl.BlockSpec((br, d), lambda i: (i, 0)),
            pl.BlockSpec((d, h), lambda i: (0, 0)),
        ],
        out_specs=pl.BlockSpec((br, h), lambda i: (i, 0)),
        out_shape=jax.ShapeDtypeStruct((np_, h), jnp.float32),
    )(deg2d, x, W_gcn)

    accs = _make_scat(np_, h, nb, cb, k)(srcr, dstr, hp)

    cst = lambda shape: pl.BlockSpec(shape, lambda i: tuple(0 for _ in shape))
    out = pl.pallas_call(
        functools.partial(_fin_body, nblk, n),
        grid=(nblk,),
        in_specs=[
            pl.BlockSpec((NC, dr, 128), lambda i: (0, i, 0)),
            pl.BlockSpec((NC, br, h), lambda i: (0, i, 0)),
            pl.BlockSpec((br, h), lambda i: (i, 0)),
            pl.BlockSpec((br, h), lambda i: (i, 0)),
            cst((b, c)),
            cst((c, h)),
            cst((1, h)),
            cst((c, h)),
            cst((1, h)),
            cst((1, h)),
            cst((1, h)),
            cst((h, 2)),
            cst((1, 2)),
        ],
        out_specs=pl.BlockSpec((b, 2), lambda i: (0, 0)),
        out_shape=jax.ShapeDtypeStruct((b, 2), jnp.float32),
        scratch_shapes=[pltpu.VMEM((h, h), jnp.float32)],
        compiler_params=pltpu.CompilerParams(
            dimension_semantics=("arbitrary",)),
    )(deg2d, accs, hp, W_proj, condition, W_gamma, b_gamma.reshape(1, h),
      W_beta, b_beta.reshape(1, h), b_gcn.reshape(1, h), b_proj.reshape(1, h),
      W_out, b_out.reshape(1, 2))
    return out
